# baseline (speedup 1.0000x reference)
import functools
import jax
import jax.numpy as jnp
from jax import lax
from jax.experimental import pallas as pl
from jax.experimental.pallas import tpu as pltpu
from jax.experimental.pallas import tpu_sc as plsc

N_NODES = 4096
NC = 2
NS = 16
NW = NC * NS
CHUNK = 128
LANES = 16
SUB = CHUNK // LANES


@functools.lru_cache(maxsize=None)
def _make_sc_diag(nrow):
    e_per_w = nrow * CHUNK
    mesh = plsc.VectorSubcoreMesh(
        core_axis_name="c", subcore_axis_name="s",
        num_cores=NC, num_subcores=NS)

    @functools.partial(
        pl.kernel,
        out_type=jax.ShapeDtypeStruct((NC, N_NODES), jnp.float32),
        mesh=mesh,
        scratch_types=[
            pltpu.VMEM((nrow, 2, CHUNK), jnp.int32),
            pltpu.VMEM((e_per_w,), jnp.int32),
            pltpu.VMEM((e_per_w,), jnp.int32),
            pltpu.VMEM((e_per_w,), jnp.float32),
            pltpu.VMEM((e_per_w,), jnp.float32),
            pltpu.VMEM((e_per_w,), jnp.float32),
            pltpu.VMEM_SHARED((N_NODES,), jnp.float32),
            pltpu.VMEM((N_NODES,), jnp.float32),
            pltpu.SemaphoreType.DMA,
            pltpu.SemaphoreType.DMA,
            pltpu.SemaphoreType.DMA,
            pltpu.SemaphoreType.DMA,
        ],
    )
    def sc_diag(edges_hbm, vals_hbm, x_hbm, out_hbm,
                edges_v, idx1_v, idx2_v, vals_v, g1_v, g2_v,
                diag_sh, zeros_v, sem1, sem2, sem3, sem4):
        cid = lax.axis_index("c")
        sid = lax.axis_index("s")
        wid = sid * NC + cid

        s_edges = pltpu.async_copy(
            edges_hbm.at[pl.ds(wid * nrow, nrow)], edges_v, sem3)
        s_vals = pltpu.async_copy(
            vals_hbm.at[pl.ds(wid * e_per_w, e_per_w)], vals_v, sem4)
        s_edges.wait()

        def idx_body(j, _):
            for k in range(SUB):
                sl = pl.ds(k * LANES, LANES)
                fl = pl.ds(j * CHUNK + k * LANES, LANES)
                r = edges_v[j, 0, sl]
                c = edges_v[j, 1, sl]
                p1 = ((r >> 3) << 15) + ((c >> 7) << 10) + ((r & 7) << 7) + (c & 127)
                p2 = ((c >> 3) << 15) + ((r >> 7) << 10) + ((c & 7) << 7) + (r & 127)
                idx2_v[fl] = p2
                idx1_v[fl] = p1
            return 0
        with jax.named_scope("idx_compute"):
            lax.fori_loop(0, nrow, idx_body, 0)

        half = (e_per_w // 2) // 8 * 8
        rest = e_per_w - half
        d1 = pltpu.async_copy(
            x_hbm.at[idx1_v.at[pl.ds(0, half)]],
            g1_v.at[pl.ds(0, half)], sem1)
        d2 = pltpu.async_copy(
            x_hbm.at[idx2_v.at[pl.ds(0, half)]],
            g2_v.at[pl.ds(0, half)], sem2)
        d3 = pltpu.async_copy(
            x_hbm.at[idx1_v.at[pl.ds(half, rest)]],
            g1_v.at[pl.ds(half, rest)], sem1)
        d4 = pltpu.async_copy(
            x_hbm.at[idx2_v.at[pl.ds(half, rest)]],
            g2_v.at[pl.ds(half, rest)], sem2)

        def z_body(j, _):
            for k in range(8):
                sl = pl.ds(j * 8 * LANES + k * LANES, LANES)
                zeros_v[sl] = jnp.zeros((LANES,), jnp.float32)
            return 0
        lax.fori_loop(0, N_NODES // (8 * LANES), z_body, 0)

        @pl.when(sid == 0)
        def _():
            pltpu.sync_copy(zeros_v, diag_sh)

        with jax.named_scope("gather_wait"):
            s_vals.wait()
            d1.wait()
            d2.wait()
            d3.wait()
            d4.wait()

        def c_body(j, _):
            for k in range(SUB):
                fl = pl.ds(j * CHUNK + k * LANES, LANES)
                a = g1_v[fl]
                b = g2_v[fl]
                v = vals_v[fl]
                vals_v[fl] = v * jnp.abs(a + b)
            return 0
        with jax.named_scope("contrib"):
            lax.fori_loop(0, nrow, c_body, 0)

        plsc.subcore_barrier()
        def a_body(j, _):
            pltpu.async_copy(
                vals_v.at[pl.ds(j * CHUNK, CHUNK)],
                diag_sh.at[edges_v.at[j, 0]], sem3, add=True)
            return 0
        with jax.named_scope("scatter_add"):
            lax.fori_loop(0, nrow, a_body, 0)
            pltpu.make_async_copy(
                vals_hbm.at[pl.ds(wid * e_per_w, e_per_w)], vals_v,
                sem3).wait()
        plsc.subcore_barrier()

        @pl.when(sid == 0)
        def _():
            pltpu.sync_copy(diag_sh, out_hbm.at[cid])

    return sc_diag


def _tc_outer(d_ref, w_ref, o_ref):
    w2 = jnp.broadcast_to(w_ref[...], (NC, w_ref.shape[1]))
    o = jax.lax.dot_general(
        d_ref[...], w2, (((0,), (0,)), ((), ())),
        precision=jax.lax.Precision.HIGHEST,
        preferred_element_type=jnp.float32)
    o_ref[...] = jnp.maximum(o, 0.0)


@jax.jit
def kernel(x, support_indices, support_values, W0):
    nnz = support_values.shape[0]
    e_per_w = -(-nnz // (NW * CHUNK)) * CHUNK
    nrow = e_per_w // CHUNK
    nnz_pad = e_per_w * NW
    pad = nnz_pad - nnz

    si = jnp.pad(support_indices, ((0, pad), (0, 0)))
    vals = jnp.pad(support_values, (0, pad))

    edges = si.reshape(nnz_pad // CHUNK, CHUNK, 2).transpose(0, 2, 1)
    x_flat = x.reshape(512, 8, 32, 128).transpose(0, 2, 1, 3).reshape(-1)

    diag_parts = _make_sc_diag(nrow)(edges, vals, x_flat)

    out_dim = W0.shape[1]
    return pl.pallas_call(
        _tc_outer,
        out_shape=jax.ShapeDtypeStruct((x.shape[0], out_dim), jnp.float32),
    )(diag_parts, W0)


# speedup vs baseline: 1.0456x; 1.0456x over previous
import functools
import jax
import jax.numpy as jnp
from jax import lax
from jax.experimental import pallas as pl
from jax.experimental.pallas import tpu as pltpu
from jax.experimental.pallas import tpu_sc as plsc

N_NODES = 4096
NC = 2
NS = 16
NW = NC * NS
CHUNK = 128
LANES = 16
SUB = CHUNK // LANES


@functools.lru_cache(maxsize=None)
def _make_sc_diag(nrow):
    e_per_w = nrow * CHUNK
    mesh = plsc.VectorSubcoreMesh(
        core_axis_name="c", subcore_axis_name="s",
        num_cores=NC, num_subcores=NS)

    @functools.partial(
        pl.kernel,
        out_type=jax.ShapeDtypeStruct((NC, N_NODES), jnp.float32),
        mesh=mesh,
        scratch_types=[
            pltpu.VMEM((nrow, 2, CHUNK), jnp.int32),
            pltpu.VMEM((e_per_w,), jnp.int32),
            pltpu.VMEM((e_per_w,), jnp.int32),
            pltpu.VMEM((e_per_w,), jnp.float32),
            pltpu.VMEM((e_per_w,), jnp.float32),
            pltpu.VMEM((e_per_w,), jnp.float32),
            pltpu.VMEM_SHARED((N_NODES,), jnp.float32),
            pltpu.VMEM((N_NODES,), jnp.float32),
            pltpu.SemaphoreType.DMA,
            pltpu.SemaphoreType.DMA,
            pltpu.SemaphoreType.DMA,
            pltpu.SemaphoreType.DMA,
            pltpu.SemaphoreType.DMA,
            pltpu.SemaphoreType.DMA,
        ],
    )
    def sc_diag(edges_hbm, vals_hbm, x_hbm, out_hbm,
                edges_v, idx1_v, idx2_v, vals_v, g1_v, g2_v,
                diag_sh, zeros_v, semq0, semq1, semq2, semq3, sem3, sem4):
        cid = lax.axis_index("c")
        sid = lax.axis_index("s")
        wid = sid * NC + cid

        s_edges = pltpu.async_copy(
            edges_hbm.at[pl.ds(wid * nrow, nrow)], edges_v, sem3)
        s_vals = pltpu.async_copy(
            vals_hbm.at[pl.ds(wid * e_per_w, e_per_w)], vals_v, sem4)
        s_edges.wait()

        def idx_body(j, _):
            for k in range(SUB):
                sl = pl.ds(k * LANES, LANES)
                fl = pl.ds(j * CHUNK + k * LANES, LANES)
                r = edges_v[j, 0, sl]
                c = edges_v[j, 1, sl]
                p1 = ((r >> 3) << 15) + ((c >> 7) << 10) + ((r & 7) << 7) + (c & 127)
                p2 = ((c >> 3) << 15) + ((r >> 7) << 10) + ((c & 7) << 7) + (r & 127)
                idx2_v[fl] = p2
                idx1_v[fl] = p1
            return 0

        def c_body(j, _):
            for k in range(SUB):
                fl = pl.ds(j * CHUNK + k * LANES, LANES)
                a = g1_v[fl]
                b = g2_v[fl]
                v = vals_v[fl]
                vals_v[fl] = v * jnp.abs(a + b)
            return 0

        def a_body(j, _):
            pltpu.async_copy(
                vals_v.at[pl.ds(j * CHUNK, CHUNK)],
                diag_sh.at[edges_v.at[j, 0]], sem3, add=True)
            return 0

        NQ = 4
        bounds = [q * nrow // NQ for q in range(NQ + 1)]
        qsems = [semq0, semq1, semq2, semq3]
        gdescs = []
        for q in range(NQ):
            lo, hi = bounds[q], bounds[q + 1]
            with jax.named_scope("idx_compute"):
                lax.fori_loop(lo, hi, idx_body, 0)
            e_lo, e_n = lo * CHUNK, (hi - lo) * CHUNK
            sl = pl.ds(e_lo, e_n)
            gdescs.append((
                pltpu.async_copy(x_hbm.at[idx1_v.at[sl]], g1_v.at[sl],
                                 qsems[q]),
                pltpu.async_copy(x_hbm.at[idx2_v.at[sl]], g2_v.at[sl],
                                 qsems[q]),
            ))

        def z_body(j, _):
            for k in range(8):
                sl = pl.ds(j * 8 * LANES + k * LANES, LANES)
                zeros_v[sl] = jnp.zeros((LANES,), jnp.float32)
            return 0
        lax.fori_loop(0, N_NODES // (8 * LANES), z_body, 0)

        @pl.when(sid == 0)
        def _():
            pltpu.sync_copy(zeros_v, diag_sh)
        s_vals.wait()
        plsc.subcore_barrier()

        for q in range(NQ):
            lo, hi = bounds[q], bounds[q + 1]
            d1, d2 = gdescs[q]
            with jax.named_scope("gather_wait"):
                d1.wait()
                d2.wait()
            with jax.named_scope("contrib"):
                lax.fori_loop(lo, hi, c_body, 0)
            with jax.named_scope("scatter_add"):
                lax.fori_loop(lo, hi, a_body, 0)

        with jax.named_scope("scatter_drain"):
            pltpu.make_async_copy(
                vals_hbm.at[pl.ds(wid * e_per_w, e_per_w)], vals_v,
                sem3).wait()
        plsc.subcore_barrier()

        @pl.when(sid == 0)
        def _():
            pltpu.sync_copy(diag_sh, out_hbm.at[cid])

    return sc_diag


def _tc_outer(d_ref, w_ref, o_ref):
    w2 = jnp.broadcast_to(w_ref[...], (NC, w_ref.shape[1]))
    o = jax.lax.dot_general(
        d_ref[...], w2, (((0,), (0,)), ((), ())),
        precision=jax.lax.Precision.HIGHEST,
        preferred_element_type=jnp.float32)
    o_ref[...] = jnp.maximum(o, 0.0)


@jax.jit
def kernel(x, support_indices, support_values, W0):
    nnz = support_values.shape[0]
    e_per_w = -(-nnz // (NW * CHUNK)) * CHUNK
    nrow = e_per_w // CHUNK
    nnz_pad = e_per_w * NW
    pad = nnz_pad - nnz

    si = jnp.pad(support_indices, ((0, pad), (0, 0)))
    vals = jnp.pad(support_values, (0, pad))

    edges = si.reshape(nnz_pad // CHUNK, CHUNK, 2).transpose(0, 2, 1)
    x_flat = x.reshape(512, 8, 32, 128).transpose(0, 2, 1, 3).reshape(-1)

    diag_parts = _make_sc_diag(nrow)(edges, vals, x_flat)

    out_dim = W0.shape[1]
    return pl.pallas_call(
        _tc_outer,
        out_shape=jax.ShapeDtypeStruct((x.shape[0], out_dim), jnp.float32),
    )(diag_parts, W0)
